# baseline (speedup 1.0000x reference)
import functools

import jax
import jax.numpy as jnp
from jax import lax
from jax.experimental import pallas as pl
from jax.experimental.pallas import tpu as pltpu
from jax.experimental.pallas import tpu_sc as plsc

D_MODEL = 4096
D_FF = 14336
N_EXPERTS = 8
B = 32
FF_TILE = 256
NT = D_FF // FF_TILE

_NEG_MIN = jnp.finfo(jnp.float32).min


def _gate_body(x_hbm, gates_t_hbm, mask_hbm, out_hbm, x_v, g_v, m_v, o_v):
    t = lax.axis_index("s") * 2 + lax.axis_index("c")
    pltpu.sync_copy(x_hbm.at[t], x_v)
    pltpu.sync_copy(gates_t_hbm, g_v)
    pltpu.sync_copy(mask_hbm, m_v)

    zeros = jnp.zeros((16,), jnp.float32)

    def body(j, accs):
        xv = x_v[pl.ds(j * 16, 16)]
        return tuple(
            accs[e] + xv * g_v[e, pl.ds(j * 16, 16)]
            for e in range(N_EXPERTS)
        )

    accs = lax.fori_loop(0, D_MODEL // 16, body, (zeros,) * N_EXPERTS)

    logits = []
    for a in accs:
        s = a[0]
        for k in range(1, 16):
            s = s + a[k]
        logits.append(s)

    ex0 = logits[0]
    for e in range(1, N_EXPERTS):
        ex0 = jnp.maximum(ex0, logits[e])
    masked = [jnp.where(l == ex0, _NEG_MIN, l) for l in logits]
    ex1 = masked[0]
    for e in range(1, N_EXPERTS):
        ex1 = jnp.maximum(ex1, masked[e])
    mv = m_v[...]
    c0 = jnp.float32(0.0)
    c1 = jnp.float32(0.0)
    for e in range(N_EXPERTS):
        me = mv[e]
        c0 = c0 + jnp.where(logits[e] == ex0, me, 0.0)
        c1 = c1 + jnp.where(logits[e] == ex1, me, 0.0)

    pre_v = 1.0 / (1.0 + jnp.exp(jnp.full((16,), ex1 - ex0, jnp.float32)))
    w_v = jnp.full((16,), c0, jnp.float32) * pre_v - \
        jnp.full((16,), c1, jnp.float32) * (pre_v - 1.0)
    o_v[...] = w_v
    pltpu.sync_copy(o_v, out_hbm.at[t])


_gate_sc = functools.partial(
    pl.kernel,
    mesh=plsc.VectorSubcoreMesh(core_axis_name="c", subcore_axis_name="s"),
    out_type=jax.ShapeDtypeStruct((B, 16), jnp.float32),
    scratch_types=[
        pltpu.VMEM((D_MODEL,), jnp.float32),
        pltpu.VMEM((N_EXPERTS, D_MODEL), jnp.float32),
        pltpu.VMEM((16,), jnp.float32),
        pltpu.VMEM((16,), jnp.float32),
    ],
)(_gate_body)


def _mlp_body(x_ref, w1_ref, w3_ref, w2_ref, out_ref, acc_ref):
    i = pl.program_id(0)

    @pl.when(i == 0)
    def _init():
        acc_ref[...] = jnp.zeros_like(acc_ref)

    xv = x_ref[...]
    h1 = jnp.dot(xv, w1_ref[...], preferred_element_type=jnp.float32)
    h3 = jnp.dot(xv, w3_ref[...], preferred_element_type=jnp.float32)
    g = (h1 * jax.nn.sigmoid(h1)) * h3
    acc_ref[...] += jnp.dot(g, w2_ref[...], preferred_element_type=jnp.float32)

    @pl.when(i == NT - 1)
    def _finish():
        out_ref[...] = acc_ref[...]


def _mlp(x2d, w1, w3, w2):
    return pl.pallas_call(
        _mlp_body,
        grid=(NT,),
        in_specs=[
            pl.BlockSpec((B, D_MODEL), lambda i: (0, 0)),
            pl.BlockSpec((D_MODEL, FF_TILE), lambda i: (0, i)),
            pl.BlockSpec((D_MODEL, FF_TILE), lambda i: (0, i)),
            pl.BlockSpec((FF_TILE, D_MODEL), lambda i: (i, 0)),
        ],
        out_specs=pl.BlockSpec((B, D_MODEL), lambda i: (0, 0)),
        out_shape=jax.ShapeDtypeStruct((B, D_MODEL), jnp.float32),
        scratch_shapes=[pltpu.VMEM((B, D_MODEL), jnp.float32)],
    )(x2d, w1, w3, w2)


@jax.jit
def _moe(x2d, gates, w1, w2, w3, expert_mask):
    gates_t = gates.T
    mpad = jnp.pad(expert_mask.reshape(N_EXPERTS), (0, 8))
    wts = _gate_sc(x2d, gates_t, mpad)
    mlp = _mlp(x2d, w1, w3, w2)
    return mlp * wts[:, :1]


def kernel(x, gates, w1, w2, w3, expert_mask):
    x2d = x.reshape(B, D_MODEL)
    out = _moe(x2d, gates, w1, w2, w3, expert_mask)
    return out.reshape(1, 1, B, D_MODEL)


# speedup vs baseline: 1.1109x; 1.1109x over previous
import jax
import jax.numpy as jnp
from jax import lax
from jax.experimental import pallas as pl
import jax.experimental.pallas.tpu as pltpu

D_MODEL = 4096
D_FF = 14336
N_EXPERTS = 8
B = 32
FF_TILE = 256
NT = D_FF // FF_TILE


def _moe_body(x_ref, gates_t_ref, mask_ref, w1_ref, w3_ref, w2_ref, out_ref,
              acc_ref, wgt_ref):
    i = pl.program_id(0)
    xv = x_ref[...]

    @pl.when(i == 0)
    def _gating():
        acc_ref[...] = jnp.zeros_like(acc_ref)
        logits = lax.dot_general(
            xv, gates_t_ref[...], (((1,), (1,)), ((), ())),
            preferred_element_type=jnp.float32)
        ex0 = jnp.max(logits, axis=1, keepdims=True)
        cond0 = (logits == ex0).astype(jnp.float32)
        neg_min = jnp.finfo(jnp.float32).min
        masked = jnp.where(cond0 > 0, neg_min, logits)
        ex1 = jnp.max(masked, axis=1, keepdims=True)
        cond1 = (logits == ex1).astype(jnp.float32)
        pre = 1.0 / (1.0 + jnp.exp(ex1 - ex0))
        c0 = jnp.dot(cond0, mask_ref[...], preferred_element_type=jnp.float32)
        c1 = jnp.dot(cond1, mask_ref[...], preferred_element_type=jnp.float32)
        wgt_ref[...] = c0 * pre - c1 * (pre - 1.0)

    h1 = jnp.dot(xv, w1_ref[...], preferred_element_type=jnp.float32)
    h3 = jnp.dot(xv, w3_ref[...], preferred_element_type=jnp.float32)
    g = (h1 * jax.nn.sigmoid(h1)) * h3
    acc_ref[...] += jnp.dot(g, w2_ref[...], preferred_element_type=jnp.float32)

    @pl.when(i == NT - 1)
    def _finish():
        out_ref[...] = acc_ref[...] * wgt_ref[...]


@jax.jit
def _moe(x2d, gates, w1, w2, w3, expert_mask):
    gates_t = gates.T
    return pl.pallas_call(
        _moe_body,
        grid=(NT,),
        in_specs=[
            pl.BlockSpec((B, D_MODEL), lambda i: (0, 0)),
            pl.BlockSpec((N_EXPERTS, D_MODEL), lambda i: (0, 0)),
            pl.BlockSpec((N_EXPERTS, 1), lambda i: (0, 0)),
            pl.BlockSpec((D_MODEL, FF_TILE), lambda i: (0, i)),
            pl.BlockSpec((D_MODEL, FF_TILE), lambda i: (0, i)),
            pl.BlockSpec((FF_TILE, D_MODEL), lambda i: (i, 0)),
        ],
        out_specs=pl.BlockSpec((B, D_MODEL), lambda i: (0, 0)),
        out_shape=jax.ShapeDtypeStruct((B, D_MODEL), jnp.float32),
        scratch_shapes=[
            pltpu.VMEM((B, D_MODEL), jnp.float32),
            pltpu.VMEM((B, 1), jnp.float32),
        ],
    )(x2d, gates_t, expert_mask, w1, w3, w2)


def kernel(x, gates, w1, w2, w3, expert_mask):
    x2d = x.reshape(B, D_MODEL)
    out = _moe(x2d, gates, w1, w2, w3, expert_mask)
    return out.reshape(1, 1, B, D_MODEL)
